# baseline (speedup 1.0000x reference)
import functools

import jax
import jax.numpy as jnp
import numpy as np
from jax import lax
from jax.experimental import pallas as pl
from jax.experimental.pallas import tpu as pltpu
from jax.experimental.pallas import tpu_sc as plsc

NC = 2
NS = 16
NW = NC * NS
LANES = 16


def _pe_row(seq_len, d_model):
    j = np.arange(d_model, dtype=np.float32)
    angle = np.float32(seq_len) / np.power(np.float32(10000.0),
                                           2.0 * j / np.float32(d_model))
    pe = np.where(np.arange(d_model) % 2 == 0, np.sin(angle), np.cos(angle))
    return jnp.asarray(pe, dtype=jnp.float32)


@jax.jit
def _sc_embed(idx, t128, pe):
    (n,) = idx.shape
    _, d2 = t128.shape
    d = d2 // 2
    assert n % NW == 0
    rows_per_w = n // NW
    chunk = 256
    assert rows_per_w % chunk == 0
    n_chunks = rows_per_w // chunk
    gsz = 128
    n_gath = chunk // gsz
    n_pe = d // LANES

    mesh = plsc.VectorSubcoreMesh(core_axis_name="c", subcore_axis_name="s",
                                  num_cores=NC, num_subcores=NS)

    @functools.partial(
        pl.kernel,
        out_type=jax.ShapeDtypeStruct((n, d), jnp.float32),
        mesh=mesh,
        scratch_types=[
            pltpu.VMEM((rows_per_w,), jnp.int32),
            pltpu.VMEM((chunk,), jnp.int32),
            pltpu.VMEM((chunk, d2), jnp.float32),
            pltpu.VMEM((chunk, d), jnp.float32),
            pltpu.VMEM((d,), jnp.float32),
            pltpu.SemaphoreType.DMA,
        ],
        compiler_params=pltpu.CompilerParams(needs_layout_passes=False),
    )
    def k(idx_hbm, t128_hbm, pe_hbm, out_hbm,
          idx_v, pidx_v, rows_v, sel_v, pe_v, sem):
        wid = lax.axis_index("s") * NC + lax.axis_index("c")
        base = wid * rows_per_w
        pltpu.sync_copy(idx_hbm.at[pl.ds(base, rows_per_w)], idx_v)
        pltpu.sync_copy(pe_hbm, pe_v)
        pe_regs = [pe_v[pl.ds(LANES * j, LANES)] for j in range(n_pe)]

        def chunk_body(c, carry):
            off = c * chunk
            def pidx_body(i, pcarry):
                v = idx_v[pl.ds(off + i * LANES, LANES)]
                pidx_v[pl.ds(i * LANES, LANES)] = jnp.right_shift(v, 1)
                return pcarry
            lax.fori_loop(0, chunk // LANES, pidx_body, 0, unroll=4)

            copies = [
                pltpu.async_copy(
                    t128_hbm.at[pidx_v.at[pl.ds(g * gsz, gsz)]],
                    rows_v.at[pl.ds(g * gsz, gsz)],
                    sem,
                )
                for g in range(n_gath)
            ]
            for cp in copies:
                cp.wait()

            def sel_body(i, scarry):
                rbase = i * LANES
                row_ids = lax.broadcasted_iota(jnp.int32, (LANES,), 0) + rbase
                par = jnp.bitwise_and(idx_v[pl.ds(off + rbase, LANES)], 1)
                col0 = par * d
                for j in range(d):
                    vals = plsc.load_gather(rows_v, [row_ids, col0 + j])
                    plsc.store_scatter(
                        sel_v, [row_ids, jnp.full((LANES,), j, jnp.int32)],
                        vals)
                return scarry

            lax.fori_loop(0, chunk // LANES, sel_body, 0)

            def pe_body(i, pcarry):
                for j in range(n_pe):
                    plsc.addupdate(sel_v.at[i, pl.ds(LANES * j, LANES)],
                                   pe_regs[j])
                return pcarry
            lax.fori_loop(0, chunk, pe_body, 0, unroll=2)

            pltpu.sync_copy(sel_v, out_hbm.at[pl.ds(base + off, chunk)])
            return carry

        lax.fori_loop(0, n_chunks, chunk_body, 0)

    return k(idx, t128, pe)


def kernel(x, table):
    b, l = x.shape
    v, d = table.shape
    idx = x.reshape(-1).astype(jnp.int32)
    t128 = table.reshape(v // 2, 2 * d)
    pe = _pe_row(l, d)
    out = _sc_embed(idx, t128, pe)
    return out.reshape(b, l, d)


# speedup vs baseline: 1.5493x; 1.5493x over previous
import functools

import jax
import jax.numpy as jnp
import numpy as np
from jax import lax
from jax.experimental import pallas as pl
from jax.experimental.pallas import tpu as pltpu
from jax.experimental.pallas import tpu_sc as plsc

NC = 2
NS = 16
NW = NC * NS
LANES = 16

MAX_SEQ_LEN = 256


def _pe_row(seq_len, d_model):
    j = np.arange(d_model, dtype=np.float32)
    angle = np.float32(seq_len) / np.power(np.float32(10000.0),
                                           2.0 * j / np.float32(d_model))
    pe = np.where(np.arange(d_model) % 2 == 0, np.sin(angle), np.cos(angle))
    return jnp.asarray(pe, dtype=jnp.float32)


@functools.partial(jax.jit, static_argnames=())
def _sc_embed(idx, table, pe):
    (n,) = idx.shape
    v, d = table.shape
    assert n % NW == 0
    rows_per_w = n // NW
    chunk = 640
    assert rows_per_w % chunk == 0
    n_chunks = rows_per_w // chunk
    gsz = 128
    n_gath = chunk // gsz
    n_pe = d // LANES

    mesh = plsc.VectorSubcoreMesh(core_axis_name="c", subcore_axis_name="s",
                                  num_cores=NC, num_subcores=NS)

    @functools.partial(
        pl.kernel,
        out_type=jax.ShapeDtypeStruct((n, d), jnp.float32),
        mesh=mesh,
        scratch_types=[
            pltpu.VMEM((rows_per_w,), jnp.int32),
            pltpu.VMEM((chunk, d), jnp.float32),
            pltpu.VMEM((d,), jnp.float32),
            pltpu.SemaphoreType.DMA,
        ],
        compiler_params=pltpu.CompilerParams(use_tc_tiling_on_sc=False),
    )
    def k(idx_hbm, table_hbm, pe_hbm, out_hbm, idx_v, rows_v, pe_v, sem):
        wid = lax.axis_index("s") * NC + lax.axis_index("c")
        base = wid * rows_per_w
        pltpu.sync_copy(idx_hbm.at[pl.ds(base, rows_per_w)], idx_v)
        pltpu.sync_copy(pe_hbm, pe_v)
        pe_regs = [pe_v[pl.ds(LANES * j, LANES)] for j in range(n_pe)]

        def chunk_body(c, carry):
            off = c * chunk
            copies = [
                pltpu.async_copy(
                    table_hbm.at[idx_v.at[pl.ds(off + g * gsz, gsz)]],
                    rows_v.at[pl.ds(g * gsz, gsz)],
                    sem,
                )
                for g in range(n_gath)
            ]
            for cp in copies:
                cp.wait()

            def row_body(i, rcarry):
                for j in range(n_pe):
                    plsc.addupdate(rows_v.at[i, pl.ds(LANES * j, LANES)],
                                   pe_regs[j])
                return rcarry

            lax.fori_loop(0, chunk, row_body, 0, unroll=2)
            pltpu.sync_copy(rows_v, out_hbm.at[pl.ds(base + off, chunk)])
            return carry

        lax.fori_loop(0, n_chunks, chunk_body, 0)

    return k(idx, table, pe)


def kernel(x, table):
    b, l = x.shape
    _, d = table.shape
    idx = x.reshape(-1).astype(jnp.int32)
    pe = _pe_row(l, d)
    out = _sc_embed(idx, table, pe)
    return out.reshape(b, l, d)


# speedup vs baseline: 1.5852x; 1.0232x over previous
import functools

import jax
import jax.numpy as jnp
import numpy as np
from jax import lax
from jax.experimental import pallas as pl
from jax.experimental.pallas import tpu as pltpu
from jax.experimental.pallas import tpu_sc as plsc

NC = 2
NS = 16
NW = NC * NS
LANES = 16

MAX_SEQ_LEN = 256


def _pe_row(seq_len, d_model):
    j = np.arange(d_model, dtype=np.float32)
    angle = np.float32(seq_len) / np.power(np.float32(10000.0),
                                           2.0 * j / np.float32(d_model))
    pe = np.where(np.arange(d_model) % 2 == 0, np.sin(angle), np.cos(angle))
    return jnp.asarray(pe, dtype=jnp.float32)


@functools.partial(jax.jit, static_argnames=())
def _sc_embed(idx, table, pe):
    (n,) = idx.shape
    v, d = table.shape
    assert n % NW == 0
    rows_per_w = n // NW
    chunk = 640
    assert rows_per_w % chunk == 0
    n_chunks = rows_per_w // chunk
    gsz = 128
    n_gath = chunk // gsz
    n_pe = d // LANES

    mesh = plsc.VectorSubcoreMesh(core_axis_name="c", subcore_axis_name="s",
                                  num_cores=NC, num_subcores=NS)

    @functools.partial(
        pl.kernel,
        out_type=jax.ShapeDtypeStruct((n, d), jnp.float32),
        mesh=mesh,
        scratch_types=[
            pltpu.VMEM((rows_per_w,), jnp.int32),
            pltpu.VMEM((chunk, d), jnp.float32),
            pltpu.VMEM((chunk, d), jnp.float32),
            pltpu.VMEM((d,), jnp.float32),
            pltpu.SemaphoreType.DMA,
            pltpu.SemaphoreType.DMA,
        ],
        compiler_params=pltpu.CompilerParams(use_tc_tiling_on_sc=False),
    )
    def k(idx_hbm, table_hbm, pe_hbm, out_hbm,
          idx_v, rows0_v, rows1_v, pe_v, sem0, sem1):
        wid = lax.axis_index("s") * NC + lax.axis_index("c")
        base = wid * rows_per_w
        pltpu.sync_copy(idx_hbm.at[pl.ds(base, rows_per_w)], idx_v)
        pltpu.sync_copy(pe_hbm, pe_v)
        pe_regs = [pe_v[pl.ds(LANES * j, LANES)] for j in range(n_pe)]

        def fire(c, buf, sem):
            off = c * chunk
            for g in range(n_gath):
                pltpu.async_copy(
                    table_hbm.at[idx_v.at[pl.ds(off + g * gsz, gsz)]],
                    buf.at[pl.ds(g * gsz, gsz)],
                    sem,
                )

        def drain(buf, sem):
            pltpu.make_async_copy(out_hbm.at[pl.ds(0, chunk)], buf, sem).wait()

        def process(c, buf):
            def row_body(i, rcarry):
                for j in range(n_pe):
                    plsc.addupdate(buf.at[i, pl.ds(LANES * j, LANES)],
                                   pe_regs[j])
                return rcarry
            lax.fori_loop(0, chunk, row_body, 0, unroll=2)
            pltpu.sync_copy(buf, out_hbm.at[pl.ds(base + c * chunk, chunk)])

        fire(0, rows0_v, sem0)

        def pair_body(c2, carry):
            c = 2 * c2
            fire(c + 1, rows1_v, sem1)
            drain(rows0_v, sem0)
            process(c, rows0_v)

            @pl.when(c2 < n_chunks // 2 - 1)
            def _():
                fire(c + 2, rows0_v, sem0)

            drain(rows1_v, sem1)
            process(c + 1, rows1_v)
            return carry

        lax.fori_loop(0, n_chunks // 2, pair_body, 0)

    return k(idx, table, pe)


def kernel(x, table):
    b, l = x.shape
    _, d = table.shape
    idx = x.reshape(-1).astype(jnp.int32)
    pe = _pe_row(l, d)
    out = _sc_embed(idx, table, pe)
    return out.reshape(b, l, d)
